# Initial kernel scaffold; baseline (speedup 1.0000x reference)
import jax
import jax.numpy as jnp
from jax.experimental import pallas as pl


def kernel(tensor_list, other, index):
    raise NotImplementedError("write your pallas kernel here")


# speedup vs baseline: 3.2651x; 3.2651x over previous
import functools

import jax
import jax.numpy as jnp
from jax import lax
from jax.experimental import pallas as pl
from jax.experimental.pallas import tpu as pltpu
from jax.experimental.pallas import tpu_sc as plsc

L = 10
B = 2
N = 2048
D = 1024
I = 256
NPAIR = B * L
NW = 32
RPW = I // NW
LANES = 16
DCH = D // LANES


def _sc_body(table_hbm, idx_hbm, bias_hbm, out_hbm, idx_vm, bias_vm,
             rows_vm, sem):
    wid = lax.axis_index("s") * 2 + lax.axis_index("c")

    pltpu.sync_copy(idx_hbm.at[wid], idx_vm)
    pltpu.sync_copy(bias_hbm, bias_vm)

    for p in range(NPAIR):
        pltpu.async_copy(table_hbm.at[idx_vm.at[p]], rows_vm, sem).wait()

        def add_row(r, _):
            def add_chunk(j, _):
                sl = pl.ds(j * LANES, LANES)
                rows_vm[r, sl] = rows_vm[r, sl] + bias_vm[p, sl]
                return 0
            lax.fori_loop(0, DCH, add_chunk, 0)
            return 0
        lax.fori_loop(0, RPW, add_row, 0)

        pltpu.sync_copy(rows_vm, out_hbm.at[pl.ds(p * I + wid * RPW, RPW)])


@jax.jit
def _run(table, idx_w, bias):
    grid_kernel = functools.partial(
        pl.kernel,
        out_type=jax.ShapeDtypeStruct((NPAIR * I, D), jnp.float32),
        mesh=plsc.VectorSubcoreMesh(core_axis_name="c", subcore_axis_name="s"),
        scratch_types=[
            pltpu.VMEM((NPAIR, RPW), jnp.int32),
            pltpu.VMEM((NPAIR, D), jnp.float32),
            pltpu.VMEM((RPW, D), jnp.float32),
            pltpu.SemaphoreType.DMA,
        ],
    )
    return grid_kernel(_sc_body)(table, idx_w, bias)


def kernel(tensor_list, other, index):
    table = tensor_list.reshape(L * B * N, D)

    b_ids = jnp.arange(B, dtype=jnp.int32)
    l_ids = jnp.arange(L, dtype=jnp.int32)
    pair_base = (l_ids[None, :] * B + b_ids[:, None]).reshape(NPAIR) * N
    idx_q = pair_base[:, None] + index[None, :].astype(jnp.int32)
    idx_w = idx_q.reshape(NPAIR, NW, RPW).transpose(1, 0, 2)

    bias = other.reshape(NPAIR, D)

    out = _run(table, idx_w, bias)
    return out.reshape(B, L, I, D)


# speedup vs baseline: 6.5463x; 2.0049x over previous
import functools

import jax
import jax.numpy as jnp
from jax import lax
from jax.experimental import pallas as pl
from jax.experimental.pallas import tpu as pltpu
from jax.experimental.pallas import tpu_sc as plsc

L = 10
B = 2
N = 2048
D = 1024
I = 256
NPAIR = B * L
NW = 32
RPW = I // NW
LANES = 16
DCH = D // LANES


def _sc_body(table_hbm, idx_hbm, bias_hbm, out_hbm, idx_vm, bias_vm,
             rows_vm, gsem, ssem):
    wid = lax.axis_index("s") * 2 + lax.axis_index("c")

    pltpu.sync_copy(idx_hbm.at[wid], idx_vm)
    pltpu.sync_copy(bias_hbm, bias_vm)

    def start_gather(p):
        return pltpu.async_copy(table_hbm.at[idx_vm.at[p]],
                                rows_vm.at[p % 2], gsem.at[p % 2])

    def start_store(p):
        return pltpu.async_copy(rows_vm.at[p % 2],
                                out_hbm.at[pl.ds(p * I + wid * RPW, RPW)],
                                ssem.at[p % 2])

    gathers = [None] * NPAIR
    stores = [None] * NPAIR
    gathers[0] = start_gather(0)
    for p in range(NPAIR):
        if p + 1 < NPAIR:
            if p >= 1:
                stores[p - 1].wait()
            gathers[p + 1] = start_gather(p + 1)
        gathers[p].wait()

        buf = rows_vm.at[p % 2]

        def add_chunk(j, _):
            sl = pl.ds(j * LANES, LANES)
            bv = bias_vm[p, sl]
            for r in range(RPW):
                buf[r, sl] = buf[r, sl] + bv
            return 0
        lax.fori_loop(0, DCH, add_chunk, 0)

        stores[p] = start_store(p)
    stores[NPAIR - 2].wait()
    stores[NPAIR - 1].wait()


@jax.jit
def _run(table, idx_w, bias):
    grid_kernel = functools.partial(
        pl.kernel,
        out_type=jax.ShapeDtypeStruct((NPAIR * I, D), jnp.float32),
        mesh=plsc.VectorSubcoreMesh(core_axis_name="c", subcore_axis_name="s"),
        scratch_types=[
            pltpu.VMEM((NPAIR, RPW), jnp.int32),
            pltpu.VMEM((NPAIR, D), jnp.float32),
            pltpu.VMEM((2, RPW, D), jnp.float32),
            pltpu.SemaphoreType.DMA((2,)),
            pltpu.SemaphoreType.DMA((2,)),
        ],
    )
    return grid_kernel(_sc_body)(table, idx_w, bias)


def kernel(tensor_list, other, index):
    table = tensor_list.reshape(L * B * N, D)

    b_ids = jnp.arange(B, dtype=jnp.int32)
    l_ids = jnp.arange(L, dtype=jnp.int32)
    pair_base = (l_ids[None, :] * B + b_ids[:, None]).reshape(NPAIR) * N
    idx_q = pair_base[:, None] + index[None, :].astype(jnp.int32)
    idx_w = idx_q.reshape(NPAIR, NW, RPW).transpose(1, 0, 2)

    bias = other.reshape(NPAIR, D)

    out = _run(table, idx_w, bias)
    return out.reshape(B, L, I, D)


# speedup vs baseline: 7.0719x; 1.0803x over previous
import functools

import jax
import jax.numpy as jnp
from jax import lax
from jax.experimental import pallas as pl
from jax.experimental.pallas import tpu as pltpu
from jax.experimental.pallas import tpu_sc as plsc

L = 10
B = 2
N = 2048
D = 1024
I = 256
NPAIR = B * L
NW = 32
RPW = I // NW
LANES = 16
DCH = D // LANES


def _sc_body(table_hbm, idx_hbm, bias_hbm, out_hbm, idx_vm, bias_vm,
             rows_vm, gsem, ssem):
    wid = lax.axis_index("s") * 2 + lax.axis_index("c")

    pltpu.sync_copy(idx_hbm.at[wid], idx_vm)
    pltpu.sync_copy(bias_hbm, bias_vm)

    def start_gather(p):
        return pltpu.async_copy(table_hbm.at[idx_vm.at[p]],
                                rows_vm.at[p % 2], gsem.at[p % 2])

    def start_store(p):
        return pltpu.async_copy(rows_vm.at[p % 2],
                                out_hbm.at[pl.ds(p * I + wid * RPW, RPW)],
                                ssem.at[p % 2])

    gathers = [None] * NPAIR
    stores = [None] * NPAIR
    gathers[0] = start_gather(0)
    for p in range(NPAIR):
        if p + 1 < NPAIR:
            if p >= 1:
                stores[p - 1].wait()
            gathers[p + 1] = start_gather(p + 1)
        gathers[p].wait()

        buf = rows_vm.at[p % 2]

        @plsc.parallel_loop(0, DCH, unroll=2)
        def _(j):
            sl = pl.ds(j * LANES, LANES)
            bv = bias_vm[p, sl]
            for r in range(RPW):
                buf[r, sl] = buf[r, sl] + bv

        stores[p] = start_store(p)
    stores[NPAIR - 2].wait()
    stores[NPAIR - 1].wait()


@jax.jit
def _run(table, idx_w, bias):
    grid_kernel = functools.partial(
        pl.kernel,
        out_type=jax.ShapeDtypeStruct((NPAIR * I, D), jnp.float32),
        mesh=plsc.VectorSubcoreMesh(core_axis_name="c", subcore_axis_name="s"),
        scratch_types=[
            pltpu.VMEM((NPAIR, RPW), jnp.int32),
            pltpu.VMEM((NPAIR, D), jnp.float32),
            pltpu.VMEM((2, RPW, D), jnp.float32),
            pltpu.SemaphoreType.DMA((2,)),
            pltpu.SemaphoreType.DMA((2,)),
        ],
    )
    return grid_kernel(_sc_body)(table, idx_w, bias)


def kernel(tensor_list, other, index):
    table = tensor_list.reshape(L * B * N, D)

    b_ids = jnp.arange(B, dtype=jnp.int32)
    l_ids = jnp.arange(L, dtype=jnp.int32)
    pair_base = (l_ids[None, :] * B + b_ids[:, None]).reshape(NPAIR) * N
    idx_q = pair_base[:, None] + index[None, :].astype(jnp.int32)
    idx_w = idx_q.reshape(NPAIR, NW, RPW).transpose(1, 0, 2)

    bias = other.reshape(NPAIR, D)

    out = _run(table, idx_w, bias)
    return out.reshape(B, L, I, D)


# speedup vs baseline: 7.3985x; 1.0462x over previous
import functools

import jax
import jax.numpy as jnp
from jax import lax
from jax.experimental import pallas as pl
from jax.experimental.pallas import tpu as pltpu
from jax.experimental.pallas import tpu_sc as plsc

L = 10
B = 2
N = 2048
D = 1024
I = 256
NPAIR = B * L
NW = 32
RPW = I // NW
LANES = 16
DCH = D // LANES


CH = 4
NCH = NPAIR // CH


def _sc_body(table_hbm, idx_hbm, bias_hbm, out_hbm, idx_vm, bias_vm,
             rows_vm, gsem, ssem):
    wid = lax.axis_index("s") * 2 + lax.axis_index("c")

    pltpu.sync_copy(idx_hbm.at[wid], idx_vm)
    pltpu.sync_copy(bias_hbm, bias_vm)

    def start_gather(c):
        return pltpu.async_copy(table_hbm.at[idx_vm.at[c]],
                                rows_vm.at[c % 2], gsem.at[c % 2])

    def start_store(c, sp):
        return pltpu.async_copy(
            rows_vm.at[c % 2].at[pl.ds(sp * RPW, RPW)],
            out_hbm.at[pl.ds((c * CH + sp) * I + wid * RPW, RPW)],
            ssem.at[c % 2])

    gathers = [None] * NCH
    stores = [None] * NCH
    gathers[0] = start_gather(0)
    for c in range(NCH):
        if c + 1 < NCH:
            if c >= 1:
                for h in stores[c - 1]:
                    h.wait()
            gathers[c + 1] = start_gather(c + 1)
        gathers[c].wait()

        buf = rows_vm.at[c % 2]
        stores[c] = []
        for sp in range(CH):
            p = c * CH + sp

            @plsc.parallel_loop(0, DCH, unroll=4)
            def _(j):
                sl = pl.ds(j * LANES, LANES)
                bv = bias_vm[p, sl]
                for r in range(RPW):
                    buf[sp * RPW + r, sl] = buf[sp * RPW + r, sl] + bv

            stores[c].append(start_store(c, sp))
    for c in (NCH - 2, NCH - 1):
        for h in stores[c]:
            h.wait()


@jax.jit
def _run(table, idx_w, bias):
    grid_kernel = functools.partial(
        pl.kernel,
        out_type=jax.ShapeDtypeStruct((NPAIR * I, D), jnp.float32),
        mesh=plsc.VectorSubcoreMesh(core_axis_name="c", subcore_axis_name="s"),
        scratch_types=[
            pltpu.VMEM((NCH, CH * RPW), jnp.int32),
            pltpu.VMEM((NPAIR, D), jnp.float32),
            pltpu.VMEM((2, CH * RPW, D), jnp.float32),
            pltpu.SemaphoreType.DMA((2,)),
            pltpu.SemaphoreType.DMA((2,)),
        ],
    )
    return grid_kernel(_sc_body)(table, idx_w, bias)


def kernel(tensor_list, other, index):
    table = tensor_list.reshape(L * B * N, D)

    b_ids = jnp.arange(B, dtype=jnp.int32)
    l_ids = jnp.arange(L, dtype=jnp.int32)
    pair_base = (l_ids[None, :] * B + b_ids[:, None]).reshape(NPAIR) * N
    idx_q = pair_base[:, None] + index[None, :].astype(jnp.int32)
    idx_w = idx_q.reshape(NPAIR, NW, RPW).transpose(1, 0, 2)
    idx_w = idx_w.reshape(NW, NCH, CH * RPW)

    bias = other.reshape(NPAIR, D)

    out = _run(table, idx_w, bias)
    return out.reshape(B, L, I, D)


# speedup vs baseline: 7.5568x; 1.0214x over previous
import functools

import jax
import jax.numpy as jnp
from jax import lax
from jax.experimental import pallas as pl
from jax.experimental.pallas import tpu as pltpu
from jax.experimental.pallas import tpu_sc as plsc

L = 10
B = 2
N = 2048
D = 1024
I = 256
NPAIR = B * L
NW = 32
RPW = I // NW
LANES = 16
DCH = D // LANES


CH = 2
NCH = NPAIR // CH
NBUF = 6


def _sc_body(table_hbm, idx_hbm, bias_hbm, out_hbm, idx_vm, bias_vm,
             rows_vm, gsem, ssem):
    wid = lax.axis_index("s") * 2 + lax.axis_index("c")

    pltpu.sync_copy(idx_hbm.at[wid], idx_vm)
    pltpu.sync_copy(bias_hbm, bias_vm)

    def start_gather(c):
        return pltpu.async_copy(table_hbm.at[idx_vm.at[c]],
                                rows_vm.at[c % NBUF], gsem.at[c % NBUF])

    def start_store(c, sp):
        return pltpu.async_copy(
            rows_vm.at[c % NBUF].at[pl.ds(sp * RPW, RPW)],
            out_hbm.at[pl.ds((c * CH + sp) * I + wid * RPW, RPW)],
            ssem.at[c % NBUF])

    gathers = [None] * NCH
    stores = [[] for _ in range(NCH)]
    for c in range(min(NBUF - 1, NCH)):
        gathers[c] = start_gather(c)
    for c in range(NCH):
        gathers[c].wait()
        buf = rows_vm.at[c % NBUF]
        for sp in range(CH):
            p = c * CH + sp

            @plsc.parallel_loop(0, DCH, unroll=4)
            def _(j):
                sl = pl.ds(j * LANES, LANES)
                bv = bias_vm[p, sl]
                for r in range(RPW):
                    buf[sp * RPW + r, sl] = buf[sp * RPW + r, sl] + bv

            stores[c].append(start_store(c, sp))
        if c + NBUF - 1 < NCH:
            if c >= 1:
                for h in stores[c - 1]:
                    h.wait()
            gathers[c + NBUF - 1] = start_gather(c + NBUF - 1)
    for c in range(max(0, NCH - NBUF), NCH):
        for h in stores[c]:
            h.wait()


@jax.jit
def _run(table, idx_w, bias):
    grid_kernel = functools.partial(
        pl.kernel,
        out_type=jax.ShapeDtypeStruct((NPAIR * I, D), jnp.float32),
        mesh=plsc.VectorSubcoreMesh(core_axis_name="c", subcore_axis_name="s"),
        scratch_types=[
            pltpu.VMEM((NCH, CH * RPW), jnp.int32),
            pltpu.VMEM((NPAIR, D), jnp.float32),
            pltpu.VMEM((NBUF, CH * RPW, D), jnp.float32),
            pltpu.SemaphoreType.DMA((NBUF,)),
            pltpu.SemaphoreType.DMA((NBUF,)),
        ],
    )
    return grid_kernel(_sc_body)(table, idx_w, bias)


def kernel(tensor_list, other, index):
    table = tensor_list.reshape(L * B * N, D)

    b_ids = jnp.arange(B, dtype=jnp.int32)
    l_ids = jnp.arange(L, dtype=jnp.int32)
    pair_base = (l_ids[None, :] * B + b_ids[:, None]).reshape(NPAIR) * N
    idx_q = pair_base[:, None] + index[None, :].astype(jnp.int32)
    idx_w = idx_q.reshape(NPAIR, NW, RPW).transpose(1, 0, 2)
    idx_w = idx_w.reshape(NW, NCH, CH * RPW)

    bias = other.reshape(NPAIR, D)

    out = _run(table, idx_w, bias)
    return out.reshape(B, L, I, D)


# speedup vs baseline: 7.6593x; 1.0136x over previous
import functools

import jax
import jax.numpy as jnp
from jax import lax
from jax.experimental import pallas as pl
from jax.experimental.pallas import tpu as pltpu
from jax.experimental.pallas import tpu_sc as plsc

L = 10
B = 2
N = 2048
D = 1024
I = 256
NPAIR = B * L
NW = 32
RPW = I // NW
LANES = 16
DCH = D // LANES


CH = 2
NCH = NPAIR // CH
NBUF = 6


def _sc_body(table_hbm, idx_hbm, bias_hbm, out_hbm, idx_vm, bias_vm,
             rows_vm, gsem, ssem, bsem):
    wid = lax.axis_index("s") * 2 + lax.axis_index("c")

    pltpu.sync_copy(idx_hbm.at[wid], idx_vm)
    bias_cp = pltpu.async_copy(bias_hbm, bias_vm, bsem)

    def start_gather(c):
        return pltpu.async_copy(table_hbm.at[idx_vm.at[c]],
                                rows_vm.at[c % NBUF], gsem.at[c % NBUF])

    def start_store(c, sp):
        return pltpu.async_copy(
            rows_vm.at[c % NBUF].at[pl.ds(sp * RPW, RPW)],
            out_hbm.at[pl.ds((c * CH + sp) * I + wid * RPW, RPW)],
            ssem.at[c % NBUF])

    gathers = [None] * NCH
    stores = [[] for _ in range(NCH)]
    for c in range(min(NBUF - 1, NCH)):
        gathers[c] = start_gather(c)
    bias_cp.wait()
    for c in range(NCH):
        gathers[c].wait()
        buf = rows_vm.at[c % NBUF]
        for sp in range(CH):
            p = c * CH + sp

            @plsc.parallel_loop(0, DCH, unroll=4)
            def _(j):
                sl = pl.ds(j * LANES, LANES)
                bv = bias_vm[p, sl]
                for r in range(RPW):
                    buf[sp * RPW + r, sl] = buf[sp * RPW + r, sl] + bv

            stores[c].append(start_store(c, sp))
        if c + NBUF - 1 < NCH:
            if c >= 1:
                for h in stores[c - 1]:
                    h.wait()
            gathers[c + NBUF - 1] = start_gather(c + NBUF - 1)
    for c in range(max(0, NCH - NBUF), NCH):
        for h in stores[c]:
            h.wait()


@jax.jit
def _run(table, idx_w, bias):
    grid_kernel = functools.partial(
        pl.kernel,
        out_type=jax.ShapeDtypeStruct((NPAIR * I, D), jnp.float32),
        mesh=plsc.VectorSubcoreMesh(core_axis_name="c", subcore_axis_name="s"),
        scratch_types=[
            pltpu.VMEM((NCH, CH * RPW), jnp.int32),
            pltpu.VMEM((NPAIR, D), jnp.float32),
            pltpu.VMEM((NBUF, CH * RPW, D), jnp.float32),
            pltpu.SemaphoreType.DMA((NBUF,)),
            pltpu.SemaphoreType.DMA((NBUF,)),
            pltpu.SemaphoreType.DMA,
        ],
    )
    return grid_kernel(_sc_body)(table, idx_w, bias)


def kernel(tensor_list, other, index):
    table = tensor_list.reshape(L * B * N, D)

    b_ids = jnp.arange(B, dtype=jnp.int32)
    l_ids = jnp.arange(L, dtype=jnp.int32)
    pair_base = (l_ids[None, :] * B + b_ids[:, None]).reshape(NPAIR) * N
    idx_q = pair_base[:, None] + index[None, :].astype(jnp.int32)
    idx_w = idx_q.reshape(NPAIR, NW, RPW).transpose(1, 0, 2)
    idx_w = idx_w.reshape(NW, NCH, CH * RPW)

    bias = other.reshape(NPAIR, D)

    out = _run(table, idx_w, bias)
    return out.reshape(B, L, I, D)
